# Initial kernel scaffold; baseline (speedup 1.0000x reference)
import jax
import jax.numpy as jnp
from jax.experimental import pallas as pl


def kernel(x, edge_index, W1, b1, W2, b2):
    raise NotImplementedError("write your pallas kernel here")


# speedup vs baseline: 11.1965x; 11.1965x over previous
import functools

import jax
import jax.numpy as jnp
from jax import lax
from jax.experimental import pallas as pl
from jax.experimental.pallas import tpu as pltpu
from jax.experimental.pallas import tpu_sc as plsc

NC = 2
NS = 16
NW = NC * NS
K = 128
DW = 16
DG = 64
BN = 512


def _cdiv(a, b):
  return (a + b - 1) // b


def _sc_degree(np_, ch):
  mesh = plsc.VectorSubcoreMesh(core_axis_name="c", subcore_axis_name="s")
  rpt = np_ // NS

  @functools.partial(
      pl.kernel,
      out_type=jax.ShapeDtypeStruct((NC, np_, DW), jnp.float32),
      mesh=mesh,
      compiler_params=pltpu.CompilerParams(use_tc_tiling_on_sc=False),
      scratch_types=[
          pltpu.VMEM_SHARED((np_, DW), jnp.float32),
          pltpu.VMEM((K, DW), jnp.float32),
          pltpu.VMEM((K,), jnp.int32),
      ],
  )
  def deg_kernel(dst_hbm, ones_hbm, zeros_hbm, out_hbm, acc, ones_v, idx_v):
    c = lax.axis_index("c")
    s = lax.axis_index("s")
    wid = c * NS + s
    pltpu.sync_copy(zeros_hbm.at[pl.ds(s * rpt, rpt)],
                    acc.at[pl.ds(s * rpt, rpt)])
    pltpu.sync_copy(ones_hbm, ones_v)
    plsc.subcore_barrier()

    @pl.loop(0, ch)
    def _(j):
      pltpu.sync_copy(dst_hbm.at[wid, j], idx_v)
      pltpu.sync_copy(ones_v, acc.at[idx_v], add=True)

    plsc.subcore_barrier()
    pltpu.sync_copy(acc.at[pl.ds(s * rpt, rpt)],
                    out_hbm.at[c, pl.ds(s * rpt, rpt)])

  return deg_kernel


def _sc_aggregate(np_, d, ch):
  mesh = plsc.VectorSubcoreMesh(core_axis_name="c", subcore_axis_name="s")
  rpt = np_ // NS

  @functools.partial(
      pl.kernel,
      out_type=jax.ShapeDtypeStruct((NC, np_, d), jnp.float32),
      mesh=mesh,
      compiler_params=pltpu.CompilerParams(use_tc_tiling_on_sc=False),
      scratch_types=[
          pltpu.VMEM_SHARED((np_, d), jnp.float32),
          pltpu.VMEM((K, d), jnp.float32),
          pltpu.VMEM((K,), jnp.int32),
          pltpu.VMEM((K,), jnp.int32),
          pltpu.SemaphoreType.DMA,
      ],
  )
  def agg_kernel(table_hbm, src_hbm, dst_hbm, zeros_hbm, out_hbm,
                 acc, rows_v, sidx_v, didx_v, sem):
    c = lax.axis_index("c")
    s = lax.axis_index("s")
    wid = c * NS + s
    pltpu.sync_copy(zeros_hbm.at[pl.ds(s * rpt, rpt)],
                    acc.at[pl.ds(s * rpt, rpt)])
    plsc.subcore_barrier()

    @pl.loop(0, ch)
    def _(j):
      pltpu.sync_copy(src_hbm.at[wid, j], sidx_v)
      pltpu.sync_copy(dst_hbm.at[wid, j], didx_v)
      pltpu.async_copy(table_hbm.at[sidx_v], rows_v, sem).wait()
      pltpu.sync_copy(rows_v, acc.at[didx_v], add=True)

    plsc.subcore_barrier()
    pltpu.sync_copy(acc.at[pl.ds(s * rpt, rpt)],
                    out_hbm.at[c, pl.ds(s * rpt, rpt)])

  return agg_kernel


def _tc_scale_body(x_ref, t0_ref, t1_ref, xs0_ref, xs1_ref, dinv_ref):
  deg = 1.0 + t0_ref[:, 0:1] + t1_ref[:, 0:1]
  dinv = lax.rsqrt(deg)
  xs = x_ref[...] * dinv
  xs0_ref[...] = xs[:, :DG]
  xs1_ref[...] = xs[:, DG:]
  dinv_ref[...] = dinv


def _tc_dense_body(p0a_ref, p1a_ref, p0b_ref, p1b_ref, xs0_ref, xs1_ref,
                   dinv_ref, w1a_ref, w1b_ref, b1_ref, w2_ref, ys_ref):
  d = dinv_ref[...]
  z0 = d * (p0a_ref[...] + p1a_ref[...] + xs0_ref[...])
  z1 = d * (p0b_ref[...] + p1b_ref[...] + xs1_ref[...])
  h = (jnp.dot(z0, w1a_ref[...], preferred_element_type=jnp.float32) +
       jnp.dot(z1, w1b_ref[...], preferred_element_type=jnp.float32))
  h = jnp.maximum(h + b1_ref[...], 0.0)
  y = jnp.dot(h, w2_ref[...], preferred_element_type=jnp.float32)
  ys_ref[...] = d * y


def _tc_softmax_body(c_valid, q0_ref, q1_ref, ys_ref, dinv_ref, b2_ref, o_ref):
  u = dinv_ref[...] * (q0_ref[...] + q1_ref[...] + ys_ref[...]) + b2_ref[...]
  col = lax.broadcasted_iota(jnp.int32, u.shape, 1)
  valid = col < c_valid
  um = jnp.where(valid, u, -jnp.inf)
  mx = jnp.max(um, axis=1, keepdims=True)
  ex = jnp.where(valid, jnp.exp(u - mx), 0.0)
  o_ref[...] = (u - mx) - jnp.log(jnp.sum(ex, axis=1, keepdims=True))


def _row_spec(d):
  return pl.BlockSpec((BN, d), lambda i: (i, 0))


def _full_spec(r, c):
  return pl.BlockSpec((r, c), lambda i: (0, 0))


def kernel(x, edge_index, W1, b1, W2, b2):
  n, f_in = x.shape
  hid = W1.shape[1]
  c_out = W2.shape[1]
  e = edge_index.shape[1]

  np_ = _cdiv(n, NS * BN) * NS * BN
  ch = _cdiv(e, NW * K)
  ep = NW * ch * K
  d2 = _cdiv(c_out, 64) * 64

  src = edge_index[0].astype(jnp.int32)
  dst = edge_index[1].astype(jnp.int32)
  pad = ep - e
  src_p = jnp.concatenate([src, jnp.zeros((pad,), jnp.int32)]).reshape(
      NW, ch, K)
  dst_p = jnp.concatenate([dst, jnp.full((pad,), n, jnp.int32)]).reshape(
      NW, ch, K)

  x_p = jnp.zeros((np_, f_in), jnp.float32).at[:n].set(x)
  ones_rows = jnp.zeros((K, DW), jnp.float32).at[:, 0].set(1.0)
  zeros_deg = jnp.zeros((np_, DW), jnp.float32)
  zeros_g = jnp.zeros((np_, DG), jnp.float32)
  zeros_c = jnp.zeros((np_, d2), jnp.float32)
  w1a = W1[:DG]
  w1b = W1[DG:]
  w2_p = jnp.zeros((hid, d2), jnp.float32).at[:, :c_out].set(W2)
  b1_r = b1.reshape(1, hid)
  b2_p = jnp.zeros((1, d2), jnp.float32).at[0, :c_out].set(b2)

  deg_t = _sc_degree(np_, ch)(dst_p, ones_rows, zeros_deg)

  grid = (np_ // BN,)
  xs0, xs1, dinv = pl.pallas_call(
      _tc_scale_body,
      grid=grid,
      in_specs=[_row_spec(f_in), _row_spec(DW), _row_spec(DW)],
      out_specs=[_row_spec(DG), _row_spec(DG), _row_spec(1)],
      out_shape=[
          jax.ShapeDtypeStruct((np_, DG), jnp.float32),
          jax.ShapeDtypeStruct((np_, DG), jnp.float32),
          jax.ShapeDtypeStruct((np_, 1), jnp.float32),
      ],
  )(x_p, deg_t[0], deg_t[1])

  agg = _sc_aggregate(np_, DG, ch)
  pa = agg(xs0, src_p, dst_p, zeros_g)
  pb = agg(xs1, src_p, dst_p, zeros_g)

  ys = pl.pallas_call(
      _tc_dense_body,
      grid=grid,
      in_specs=[
          _row_spec(DG), _row_spec(DG), _row_spec(DG), _row_spec(DG),
          _row_spec(DG), _row_spec(DG), _row_spec(1),
          _full_spec(DG, hid), _full_spec(DG, hid), _full_spec(1, hid),
          _full_spec(hid, d2),
      ],
      out_specs=_row_spec(d2),
      out_shape=jax.ShapeDtypeStruct((np_, d2), jnp.float32),
  )(pa[0], pa[1], pb[0], pb[1], xs0, xs1, dinv, w1a, w1b, b1_r, w2_p)

  q_t = _sc_aggregate(np_, d2, ch)(ys, src_p, dst_p, zeros_c)

  o = pl.pallas_call(
      functools.partial(_tc_softmax_body, c_out),
      grid=grid,
      in_specs=[
          _row_spec(d2), _row_spec(d2), _row_spec(d2), _row_spec(1),
          _full_spec(1, d2),
      ],
      out_specs=_row_spec(d2),
      out_shape=jax.ShapeDtypeStruct((np_, d2), jnp.float32),
  )(q_t[0], q_t[1], ys, dinv, b2_p)

  return o[:n, :c_out]


# speedup vs baseline: 11.4392x; 1.0217x over previous
import functools

import jax
import jax.numpy as jnp
from jax import lax
from jax.experimental import pallas as pl
from jax.experimental.pallas import tpu as pltpu
from jax.experimental.pallas import tpu_sc as plsc

NC = 2
NS = 16
NW = NC * NS
K = 128
DW = 16
DG = 64
BN = 512


def _cdiv(a, b):
  return (a + b - 1) // b


NB = 4


def _sc_degree(np_, ch):
  mesh = plsc.VectorSubcoreMesh(core_axis_name="c", subcore_axis_name="s")
  rpt = np_ // NS

  @functools.partial(
      pl.kernel,
      out_type=jax.ShapeDtypeStruct((NC, np_, DW), jnp.float32),
      mesh=mesh,
      compiler_params=pltpu.CompilerParams(use_tc_tiling_on_sc=False),
      scratch_types=[
          pltpu.VMEM_SHARED((np_, DW), jnp.float32),
          pltpu.VMEM((K, DW), jnp.float32),
          pltpu.VMEM((ch, K), jnp.int32),
      ] + [pltpu.SemaphoreType.DMA] * NB,
  )
  def deg_kernel(dst_hbm, ones_hbm, zeros_hbm, out_hbm, acc, ones_v, didx,
                 *sems):
    c = lax.axis_index("c")
    s = lax.axis_index("s")
    wid = c * NS + s
    pltpu.sync_copy(zeros_hbm.at[pl.ds(s * rpt, rpt)],
                    acc.at[pl.ds(s * rpt, rpt)])
    pltpu.sync_copy(ones_hbm, ones_v)
    pltpu.sync_copy(dst_hbm.at[wid], didx)
    plsc.subcore_barrier()

    @pl.loop(0, ch // NB)
    def _(t):
      j0 = t * NB
      sd = [
          pltpu.async_copy(ones_v, acc.at[didx.at[j0 + b]], sems[b], add=True)
          for b in range(NB)
      ]
      for b in range(NB):
        sd[b].wait()

    plsc.subcore_barrier()
    pltpu.sync_copy(acc.at[pl.ds(s * rpt, rpt)],
                    out_hbm.at[c, pl.ds(s * rpt, rpt)])

  return deg_kernel


def _sc_aggregate(np_, d, ch):
  mesh = plsc.VectorSubcoreMesh(core_axis_name="c", subcore_axis_name="s")
  rpt = np_ // NS

  @functools.partial(
      pl.kernel,
      out_type=jax.ShapeDtypeStruct((NC, np_, d), jnp.float32),
      mesh=mesh,
      compiler_params=pltpu.CompilerParams(use_tc_tiling_on_sc=False),
      scratch_types=[
          pltpu.VMEM_SHARED((np_, d), jnp.float32),
          pltpu.VMEM((NB, K, d), jnp.float32),
          pltpu.VMEM((ch, K), jnp.int32),
          pltpu.VMEM((ch, K), jnp.int32),
      ] + [pltpu.SemaphoreType.DMA] * (2 * NB),
  )
  def agg_kernel(table_hbm, src_hbm, dst_hbm, zeros_hbm, out_hbm,
                 acc, rows, sidx, didx, *sems):
    gsems = sems[:NB]
    ssems = sems[NB:]
    c = lax.axis_index("c")
    s = lax.axis_index("s")
    wid = c * NS + s
    pltpu.sync_copy(zeros_hbm.at[pl.ds(s * rpt, rpt)],
                    acc.at[pl.ds(s * rpt, rpt)])
    pltpu.sync_copy(src_hbm.at[wid], sidx)
    pltpu.sync_copy(dst_hbm.at[wid], didx)
    plsc.subcore_barrier()

    @pl.loop(0, ch // NB)
    def _(t):
      j0 = t * NB
      gd = [
          pltpu.async_copy(table_hbm.at[sidx.at[j0 + b]], rows.at[b], gsems[b])
          for b in range(NB)
      ]
      sd = []
      for b in range(NB):
        gd[b].wait()
        sd.append(
            pltpu.async_copy(rows.at[b], acc.at[didx.at[j0 + b]], ssems[b],
                             add=True))
      for b in range(NB):
        sd[b].wait()

    plsc.subcore_barrier()
    pltpu.sync_copy(acc.at[pl.ds(s * rpt, rpt)],
                    out_hbm.at[c, pl.ds(s * rpt, rpt)])

  return agg_kernel


def _tc_scale_body(x_ref, t0_ref, t1_ref, xs0_ref, xs1_ref, dinv_ref):
  deg = 1.0 + t0_ref[:, 0:1] + t1_ref[:, 0:1]
  dinv = lax.rsqrt(deg)
  xs = x_ref[...] * dinv
  xs0_ref[...] = xs[:, :DG]
  xs1_ref[...] = xs[:, DG:]
  dinv_ref[...] = dinv


def _tc_dense_body(p0a_ref, p1a_ref, p0b_ref, p1b_ref, xs0_ref, xs1_ref,
                   dinv_ref, w1a_ref, w1b_ref, b1_ref, w2_ref, ys_ref):
  d = dinv_ref[...]
  z0 = d * (p0a_ref[...] + p1a_ref[...] + xs0_ref[...])
  z1 = d * (p0b_ref[...] + p1b_ref[...] + xs1_ref[...])
  h = (jnp.dot(z0, w1a_ref[...], preferred_element_type=jnp.float32) +
       jnp.dot(z1, w1b_ref[...], preferred_element_type=jnp.float32))
  h = jnp.maximum(h + b1_ref[...], 0.0)
  y = jnp.dot(h, w2_ref[...], preferred_element_type=jnp.float32)
  ys_ref[...] = d * y


def _tc_softmax_body(c_valid, q0_ref, q1_ref, ys_ref, dinv_ref, b2_ref, o_ref):
  u = dinv_ref[...] * (q0_ref[...] + q1_ref[...] + ys_ref[...]) + b2_ref[...]
  col = lax.broadcasted_iota(jnp.int32, u.shape, 1)
  valid = col < c_valid
  um = jnp.where(valid, u, -jnp.inf)
  mx = jnp.max(um, axis=1, keepdims=True)
  ex = jnp.where(valid, jnp.exp(u - mx), 0.0)
  o_ref[...] = (u - mx) - jnp.log(jnp.sum(ex, axis=1, keepdims=True))


def _row_spec(d):
  return pl.BlockSpec((BN, d), lambda i: (i, 0))


def _full_spec(r, c):
  return pl.BlockSpec((r, c), lambda i: (0, 0))


def kernel(x, edge_index, W1, b1, W2, b2):
  n, f_in = x.shape
  hid = W1.shape[1]
  c_out = W2.shape[1]
  e = edge_index.shape[1]

  np_ = _cdiv(n, NS * BN) * NS * BN
  ch = _cdiv(_cdiv(e, NW * K), NB) * NB
  ep = NW * ch * K
  d2 = _cdiv(c_out, 64) * 64

  src = edge_index[0].astype(jnp.int32)
  dst = edge_index[1].astype(jnp.int32)
  pad = ep - e
  src_p = jnp.concatenate([src, jnp.zeros((pad,), jnp.int32)]).reshape(
      NW, ch, K)
  dst_p = jnp.concatenate([dst, jnp.full((pad,), n, jnp.int32)]).reshape(
      NW, ch, K)

  x_p = jnp.zeros((np_, f_in), jnp.float32).at[:n].set(x)
  ones_rows = jnp.zeros((K, DW), jnp.float32).at[:, 0].set(1.0)
  zeros_deg = jnp.zeros((np_, DW), jnp.float32)
  zeros_g = jnp.zeros((np_, DG), jnp.float32)
  zeros_c = jnp.zeros((np_, d2), jnp.float32)
  w1a = W1[:DG]
  w1b = W1[DG:]
  w2_p = jnp.zeros((hid, d2), jnp.float32).at[:, :c_out].set(W2)
  b1_r = b1.reshape(1, hid)
  b2_p = jnp.zeros((1, d2), jnp.float32).at[0, :c_out].set(b2)

  deg_t = _sc_degree(np_, ch)(dst_p, ones_rows, zeros_deg)

  grid = (np_ // BN,)
  xs0, xs1, dinv = pl.pallas_call(
      _tc_scale_body,
      grid=grid,
      in_specs=[_row_spec(f_in), _row_spec(DW), _row_spec(DW)],
      out_specs=[_row_spec(DG), _row_spec(DG), _row_spec(1)],
      out_shape=[
          jax.ShapeDtypeStruct((np_, DG), jnp.float32),
          jax.ShapeDtypeStruct((np_, DG), jnp.float32),
          jax.ShapeDtypeStruct((np_, 1), jnp.float32),
      ],
  )(x_p, deg_t[0], deg_t[1])

  agg = _sc_aggregate(np_, DG, ch)
  pa = agg(xs0, src_p, dst_p, zeros_g)
  pb = agg(xs1, src_p, dst_p, zeros_g)

  ys = pl.pallas_call(
      _tc_dense_body,
      grid=grid,
      in_specs=[
          _row_spec(DG), _row_spec(DG), _row_spec(DG), _row_spec(DG),
          _row_spec(DG), _row_spec(DG), _row_spec(1),
          _full_spec(DG, hid), _full_spec(DG, hid), _full_spec(1, hid),
          _full_spec(hid, d2),
      ],
      out_specs=_row_spec(d2),
      out_shape=jax.ShapeDtypeStruct((np_, d2), jnp.float32),
  )(pa[0], pa[1], pb[0], pb[1], xs0, xs1, dinv, w1a, w1b, b1_r, w2_p)

  q_t = _sc_aggregate(np_, d2, ch)(ys, src_p, dst_p, zeros_c)

  o = pl.pallas_call(
      functools.partial(_tc_softmax_body, c_out),
      grid=grid,
      in_specs=[
          _row_spec(d2), _row_spec(d2), _row_spec(d2), _row_spec(1),
          _full_spec(1, d2),
      ],
      out_specs=_row_spec(d2),
      out_shape=jax.ShapeDtypeStruct((np_, d2), jnp.float32),
  )(q_t[0], q_t[1], ys, dinv, b2_p)

  return o[:n, :c_out]


# speedup vs baseline: 11.6540x; 1.0188x over previous
import functools

import jax
import jax.numpy as jnp
from jax import lax
from jax.experimental import pallas as pl
from jax.experimental.pallas import tpu as pltpu
from jax.experimental.pallas import tpu_sc as plsc

NC = 2
NS = 16
NW = NC * NS
K2 = 256
DW = 16
DG = 64
BN = 512


def _cdiv(a, b):
  return (a + b - 1) // b


def _sc_degree(np_, ch):
  mesh = plsc.VectorSubcoreMesh(core_axis_name="c", subcore_axis_name="s")
  rpt = np_ // NS

  @functools.partial(
      pl.kernel,
      out_type=jax.ShapeDtypeStruct((NC, np_, DW), jnp.float32),
      mesh=mesh,
      compiler_params=pltpu.CompilerParams(use_tc_tiling_on_sc=False),
      scratch_types=[
          pltpu.VMEM_SHARED((np_, DW), jnp.float32),
          pltpu.VMEM((K2, DW), jnp.float32),
          pltpu.VMEM((2, 2, K2), jnp.int32),
      ] + [pltpu.SemaphoreType.DMA] * 4,
  )
  def deg_kernel(sd_hbm, ones_hbm, zeros_hbm, out_hbm, acc, ones_v, idxv,
                 *sems):
    isems = sems[0:2]
    ssems = sems[2:4]
    c = lax.axis_index("c")
    s = lax.axis_index("s")
    wid = c * NS + s
    pltpu.sync_copy(zeros_hbm.at[pl.ds(s * rpt, rpt)],
                    acc.at[pl.ds(s * rpt, rpt)])
    pltpu.sync_copy(ones_hbm, ones_v)
    plsc.subcore_barrier()

    @pl.loop(0, ch // 2)
    def _(t):
      idd = [
          pltpu.async_copy(sd_hbm.at[wid, 2 * t + b], idxv.at[b], isems[b])
          for b in range(2)
      ]
      sd = []
      for b in range(2):
        idd[b].wait()
        sd.append(
            pltpu.async_copy(ones_v, acc.at[idxv.at[b, 1]], ssems[b],
                             add=True))
      for b in range(2):
        sd[b].wait()

    plsc.subcore_barrier()
    pltpu.sync_copy(acc.at[pl.ds(s * rpt, rpt)],
                    out_hbm.at[c, pl.ds(s * rpt, rpt)])

  return deg_kernel


def _sc_aggregate(np_, d, ch):
  mesh = plsc.VectorSubcoreMesh(core_axis_name="c", subcore_axis_name="s")
  rpt = np_ // NS

  @functools.partial(
      pl.kernel,
      out_type=jax.ShapeDtypeStruct((NC, np_, d), jnp.float32),
      mesh=mesh,
      compiler_params=pltpu.CompilerParams(use_tc_tiling_on_sc=False),
      scratch_types=[
          pltpu.VMEM_SHARED((np_, d), jnp.float32),
          pltpu.VMEM((2, K2, d), jnp.float32),
          pltpu.VMEM((2, 2, K2), jnp.int32),
      ] + [pltpu.SemaphoreType.DMA] * 6,
  )
  def agg_kernel(table_hbm, sd_hbm, zeros_hbm, out_hbm, acc, rows, idxv,
                 *sems):
    isems = sems[0:2]
    gsems = sems[2:4]
    ssems = sems[4:6]
    c = lax.axis_index("c")
    s = lax.axis_index("s")
    wid = c * NS + s
    pltpu.sync_copy(zeros_hbm.at[pl.ds(s * rpt, rpt)],
                    acc.at[pl.ds(s * rpt, rpt)])
    plsc.subcore_barrier()

    @pl.loop(0, ch // 2)
    def _(t):
      idd = [
          pltpu.async_copy(sd_hbm.at[wid, 2 * t + b], idxv.at[b], isems[b])
          for b in range(2)
      ]
      gd = []
      for b in range(2):
        idd[b].wait()
        gd.append(
            pltpu.async_copy(table_hbm.at[idxv.at[b, 0]], rows.at[b],
                             gsems[b]))
      sd = []
      for b in range(2):
        gd[b].wait()
        sd.append(
            pltpu.async_copy(rows.at[b], acc.at[idxv.at[b, 1]], ssems[b],
                             add=True))
      for b in range(2):
        sd[b].wait()

    plsc.subcore_barrier()
    pltpu.sync_copy(acc.at[pl.ds(s * rpt, rpt)],
                    out_hbm.at[c, pl.ds(s * rpt, rpt)])

  return agg_kernel


def _tc_scale_body(x_ref, t0_ref, t1_ref, xs0_ref, xs1_ref, dinv_ref):
  deg = 1.0 + t0_ref[:, 0:1] + t1_ref[:, 0:1]
  dinv = lax.rsqrt(deg)
  xs = x_ref[...] * dinv
  xs0_ref[...] = xs[:, :DG]
  xs1_ref[...] = xs[:, DG:]
  dinv_ref[...] = dinv


def _tc_dense_body(p0a_ref, p1a_ref, p0b_ref, p1b_ref, xs0_ref, xs1_ref,
                   dinv_ref, w1a_ref, w1b_ref, b1_ref, w2_ref, ys_ref):
  d = dinv_ref[...]
  z0 = d * (p0a_ref[...] + p1a_ref[...] + xs0_ref[...])
  z1 = d * (p0b_ref[...] + p1b_ref[...] + xs1_ref[...])
  h = (jnp.dot(z0, w1a_ref[...], preferred_element_type=jnp.float32) +
       jnp.dot(z1, w1b_ref[...], preferred_element_type=jnp.float32))
  h = jnp.maximum(h + b1_ref[...], 0.0)
  y = jnp.dot(h, w2_ref[...], preferred_element_type=jnp.float32)
  ys_ref[...] = d * y


def _tc_softmax_body(c_valid, q0_ref, q1_ref, ys_ref, dinv_ref, b2_ref, o_ref):
  u = dinv_ref[...] * (q0_ref[...] + q1_ref[...] + ys_ref[...]) + b2_ref[...]
  col = lax.broadcasted_iota(jnp.int32, u.shape, 1)
  valid = col < c_valid
  um = jnp.where(valid, u, -jnp.inf)
  mx = jnp.max(um, axis=1, keepdims=True)
  ex = jnp.where(valid, jnp.exp(u - mx), 0.0)
  o_ref[...] = (u - mx) - jnp.log(jnp.sum(ex, axis=1, keepdims=True))


def _row_spec(d):
  return pl.BlockSpec((BN, d), lambda i: (i, 0))


def _full_spec(r, c):
  return pl.BlockSpec((r, c), lambda i: (0, 0))


def kernel(x, edge_index, W1, b1, W2, b2):
  n, f_in = x.shape
  hid = W1.shape[1]
  c_out = W2.shape[1]
  e = edge_index.shape[1]

  np_ = _cdiv(n, NS * BN) * NS * BN
  ch = _cdiv(_cdiv(e, NW * K2), 2) * 2
  ep = NW * ch * K2
  d2 = _cdiv(c_out, 64) * 64

  src = edge_index[0].astype(jnp.int32)
  dst = edge_index[1].astype(jnp.int32)
  pad = ep - e
  src_p = jnp.concatenate([src, jnp.zeros((pad,), jnp.int32)]).reshape(
      NW, ch, K2)
  dst_p = jnp.concatenate([dst, jnp.full((pad,), n, jnp.int32)]).reshape(
      NW, ch, K2)
  sd_p = jnp.stack([src_p, dst_p], axis=2)

  x_p = jnp.zeros((np_, f_in), jnp.float32).at[:n].set(x)
  ones_rows = jnp.zeros((K2, DW), jnp.float32).at[:, 0].set(1.0)
  zeros_deg = jnp.zeros((np_, DW), jnp.float32)
  zeros_g = jnp.zeros((np_, DG), jnp.float32)
  zeros_c = jnp.zeros((np_, d2), jnp.float32)
  w1a = W1[:DG]
  w1b = W1[DG:]
  w2_p = jnp.zeros((hid, d2), jnp.float32).at[:, :c_out].set(W2)
  b1_r = b1.reshape(1, hid)
  b2_p = jnp.zeros((1, d2), jnp.float32).at[0, :c_out].set(b2)

  deg_t = _sc_degree(np_, ch)(sd_p, ones_rows, zeros_deg)

  grid = (np_ // BN,)
  xs0, xs1, dinv = pl.pallas_call(
      _tc_scale_body,
      grid=grid,
      in_specs=[_row_spec(f_in), _row_spec(DW), _row_spec(DW)],
      out_specs=[_row_spec(DG), _row_spec(DG), _row_spec(1)],
      out_shape=[
          jax.ShapeDtypeStruct((np_, DG), jnp.float32),
          jax.ShapeDtypeStruct((np_, DG), jnp.float32),
          jax.ShapeDtypeStruct((np_, 1), jnp.float32),
      ],
  )(x_p, deg_t[0], deg_t[1])

  agg = _sc_aggregate(np_, DG, ch)
  pa = agg(xs0, sd_p, zeros_g)
  pb = agg(xs1, sd_p, zeros_g)

  ys = pl.pallas_call(
      _tc_dense_body,
      grid=grid,
      in_specs=[
          _row_spec(DG), _row_spec(DG), _row_spec(DG), _row_spec(DG),
          _row_spec(DG), _row_spec(DG), _row_spec(1),
          _full_spec(DG, hid), _full_spec(DG, hid), _full_spec(1, hid),
          _full_spec(hid, d2),
      ],
      out_specs=_row_spec(d2),
      out_shape=jax.ShapeDtypeStruct((np_, d2), jnp.float32),
  )(pa[0], pa[1], pb[0], pb[1], xs0, xs1, dinv, w1a, w1b, b1_r, w2_p)

  q_t = _sc_aggregate(np_, d2, ch)(ys, sd_p, zeros_c)

  o = pl.pallas_call(
      functools.partial(_tc_softmax_body, c_out),
      grid=grid,
      in_specs=[
          _row_spec(d2), _row_spec(d2), _row_spec(d2), _row_spec(1),
          _full_spec(1, d2),
      ],
      out_specs=_row_spec(d2),
      out_shape=jax.ShapeDtypeStruct((np_, d2), jnp.float32),
  )(q_t[0], q_t[1], ys, dinv, b2_p)

  return o[:n, :c_out]


# speedup vs baseline: 16.8222x; 1.4435x over previous
import functools

import jax
import jax.numpy as jnp
from jax import lax
from jax.experimental import pallas as pl
from jax.experimental.pallas import tpu as pltpu
from jax.experimental.pallas import tpu_sc as plsc

NC = 2
NS = 16
NW = NC * NS
K2 = 256
DW = 16
DG = 64
BN = 512


def _cdiv(a, b):
  return (a + b - 1) // b


def _sc_degree(np_, ch):
  mesh = plsc.VectorSubcoreMesh(core_axis_name="c", subcore_axis_name="s")
  rpt = np_ // NS

  @functools.partial(
      pl.kernel,
      out_type=jax.ShapeDtypeStruct((NC, np_, DW), jnp.float32),
      mesh=mesh,
      compiler_params=pltpu.CompilerParams(use_tc_tiling_on_sc=False),
      scratch_types=[
          pltpu.VMEM_SHARED((np_, DW), jnp.float32),
          pltpu.VMEM((K2, DW), jnp.float32),
          pltpu.VMEM((2, 2, K2), jnp.int32),
      ] + [pltpu.SemaphoreType.DMA] * 4,
  )
  def deg_kernel(sd_hbm, ones_hbm, zeros_hbm, out_hbm, acc, ones_v, idxv,
                 *sems):
    isems = sems[0:2]
    ssems = sems[2:4]
    c = lax.axis_index("c")
    s = lax.axis_index("s")
    wid = c * NS + s
    pltpu.sync_copy(zeros_hbm.at[pl.ds(s * rpt, rpt)],
                    acc.at[pl.ds(s * rpt, rpt)])
    pltpu.sync_copy(ones_hbm, ones_v)
    plsc.subcore_barrier()

    @pl.loop(0, ch // 2)
    def _(t):
      idd = [
          pltpu.async_copy(sd_hbm.at[wid, 2 * t + b], idxv.at[b], isems[b])
          for b in range(2)
      ]
      sd = []
      for b in range(2):
        idd[b].wait()
        sd.append(
            pltpu.async_copy(ones_v, acc.at[idxv.at[b, 1]], ssems[b],
                             add=True))
      for b in range(2):
        sd[b].wait()

    plsc.subcore_barrier()
    pltpu.sync_copy(acc.at[pl.ds(s * rpt, rpt)],
                    out_hbm.at[c, pl.ds(s * rpt, rpt)])

  return deg_kernel


def _sc_aggregate(np_, d, ch, dtype):
  mesh = plsc.VectorSubcoreMesh(core_axis_name="c", subcore_axis_name="s")
  rpt = np_ // NS

  @functools.partial(
      pl.kernel,
      out_type=jax.ShapeDtypeStruct((NC, np_, d), dtype),
      mesh=mesh,
      compiler_params=pltpu.CompilerParams(use_tc_tiling_on_sc=False),
      scratch_types=[
          pltpu.VMEM_SHARED((np_, d), dtype),
          pltpu.VMEM((2, K2, d), dtype),
          pltpu.VMEM((2, 2, K2), jnp.int32),
      ] + [pltpu.SemaphoreType.DMA] * 6,
  )
  def agg_kernel(table_hbm, sd_hbm, zeros_hbm, out_hbm, acc, rows, idxv,
                 *sems):
    isems = sems[0:2]
    gsems = sems[2:4]
    ssems = sems[4:6]
    c = lax.axis_index("c")
    s = lax.axis_index("s")
    wid = c * NS + s
    pltpu.sync_copy(zeros_hbm.at[pl.ds(s * rpt, rpt)],
                    acc.at[pl.ds(s * rpt, rpt)])
    plsc.subcore_barrier()

    @pl.loop(0, ch // 2)
    def _(t):
      idd = [
          pltpu.async_copy(sd_hbm.at[wid, 2 * t + b], idxv.at[b], isems[b])
          for b in range(2)
      ]
      gd = []
      for b in range(2):
        idd[b].wait()
        gd.append(
            pltpu.async_copy(table_hbm.at[idxv.at[b, 0]], rows.at[b],
                             gsems[b]))
      sd = []
      for b in range(2):
        gd[b].wait()
        sd.append(
            pltpu.async_copy(rows.at[b], acc.at[idxv.at[b, 1]], ssems[b],
                             add=True))
      for b in range(2):
        sd[b].wait()

    plsc.subcore_barrier()
    pltpu.sync_copy(acc.at[pl.ds(s * rpt, rpt)],
                    out_hbm.at[c, pl.ds(s * rpt, rpt)])

  return agg_kernel


def _tc_scale_body(x_ref, t0_ref, t1_ref, xs_ref, dinv_ref):
  deg = 1.0 + t0_ref[:, 0:1] + t1_ref[:, 0:1]
  dinv = lax.rsqrt(deg)
  xs_ref[...] = (x_ref[...] * dinv).astype(xs_ref.dtype)
  dinv_ref[...] = dinv


def _tc_dense_body(p0_ref, p1_ref, xs_ref, dinv_ref, w1_ref, b1_ref, w2_ref,
                   ys_ref):
  d = dinv_ref[...]
  agg = (p0_ref[...].astype(jnp.float32) + p1_ref[...].astype(jnp.float32) +
         xs_ref[...].astype(jnp.float32))
  z = d * agg
  h = jnp.dot(z, w1_ref[...], preferred_element_type=jnp.float32)
  h = jnp.maximum(h + b1_ref[...], 0.0)
  y = jnp.dot(h, w2_ref[...], preferred_element_type=jnp.float32)
  ys_ref[...] = (d * y).astype(ys_ref.dtype)


def _tc_softmax_body(c_valid, q0_ref, q1_ref, ys_ref, dinv_ref, b2_ref, o_ref):
  agg = (q0_ref[...].astype(jnp.float32) + q1_ref[...].astype(jnp.float32) +
         ys_ref[...].astype(jnp.float32))
  u = dinv_ref[...] * agg + b2_ref[...]
  col = lax.broadcasted_iota(jnp.int32, u.shape, 1)
  valid = col < c_valid
  um = jnp.where(valid, u, -jnp.inf)
  mx = jnp.max(um, axis=1, keepdims=True)
  ex = jnp.where(valid, jnp.exp(u - mx), 0.0)
  o_ref[...] = (u - mx) - jnp.log(jnp.sum(ex, axis=1, keepdims=True))


def _row_spec(d):
  return pl.BlockSpec((BN, d), lambda i: (i, 0))


def _full_spec(r, c):
  return pl.BlockSpec((r, c), lambda i: (0, 0))


def kernel(x, edge_index, W1, b1, W2, b2):
  n, f_in = x.shape
  hid = W1.shape[1]
  c_out = W2.shape[1]
  e = edge_index.shape[1]

  np_ = _cdiv(n, NS * BN) * NS * BN
  ch = _cdiv(_cdiv(e, NW * K2), 2) * 2
  ep = NW * ch * K2
  d2 = _cdiv(c_out, 64) * 64

  src = edge_index[0].astype(jnp.int32)
  dst = edge_index[1].astype(jnp.int32)
  pad = ep - e
  src_p = jnp.concatenate([src, jnp.zeros((pad,), jnp.int32)]).reshape(
      NW, ch, K2)
  dst_p = jnp.concatenate([dst, jnp.full((pad,), n, jnp.int32)]).reshape(
      NW, ch, K2)
  sd_p = jnp.stack([src_p, dst_p], axis=2)

  x_p = jnp.zeros((np_, f_in), jnp.float32).at[:n].set(x)
  ones_rows = jnp.zeros((K2, DW), jnp.float32).at[:, 0].set(1.0)
  zeros_deg = jnp.zeros((np_, DW), jnp.float32)
  zeros_g = jnp.zeros((np_, f_in), jnp.bfloat16)
  zeros_c = jnp.zeros((np_, d2), jnp.bfloat16)
  w2_p = jnp.zeros((hid, d2), jnp.float32).at[:, :c_out].set(W2)
  b1_r = b1.reshape(1, hid)
  b2_p = jnp.zeros((1, d2), jnp.float32).at[0, :c_out].set(b2)

  deg_t = _sc_degree(np_, ch)(sd_p, ones_rows, zeros_deg)

  grid = (np_ // BN,)
  xs, dinv = pl.pallas_call(
      _tc_scale_body,
      grid=grid,
      in_specs=[_row_spec(f_in), _row_spec(DW), _row_spec(DW)],
      out_specs=[_row_spec(f_in), _row_spec(1)],
      out_shape=[
          jax.ShapeDtypeStruct((np_, f_in), jnp.bfloat16),
          jax.ShapeDtypeStruct((np_, 1), jnp.float32),
      ],
  )(x_p, deg_t[0], deg_t[1])

  p_t = _sc_aggregate(np_, f_in, ch, jnp.bfloat16)(xs, sd_p, zeros_g)

  ys = pl.pallas_call(
      _tc_dense_body,
      grid=grid,
      in_specs=[
          _row_spec(f_in), _row_spec(f_in), _row_spec(f_in), _row_spec(1),
          _full_spec(f_in, hid), _full_spec(1, hid), _full_spec(hid, d2),
      ],
      out_specs=_row_spec(d2),
      out_shape=jax.ShapeDtypeStruct((np_, d2), jnp.bfloat16),
  )(p_t[0], p_t[1], xs, dinv, W1, b1_r, w2_p)

  q_t = _sc_aggregate(np_, d2, ch, jnp.bfloat16)(ys, sd_p, zeros_c)

  o = pl.pallas_call(
      functools.partial(_tc_softmax_body, c_out),
      grid=grid,
      in_specs=[
          _row_spec(d2), _row_spec(d2), _row_spec(d2), _row_spec(1),
          _full_spec(1, d2),
      ],
      out_specs=_row_spec(d2),
      out_shape=jax.ShapeDtypeStruct((np_, d2), jnp.float32),
  )(q_t[0], q_t[1], ys, dinv, b2_p)

  return o[:n, :c_out]


# speedup vs baseline: 17.1592x; 1.0200x over previous
import functools

import jax
import jax.numpy as jnp
from jax import lax
from jax.experimental import pallas as pl
from jax.experimental.pallas import tpu as pltpu
from jax.experimental.pallas import tpu_sc as plsc

NC = 2
NS = 16
NW = NC * NS
K2 = 256
DW = 16
DG = 64
BN = 512


def _cdiv(a, b):
  return (a + b - 1) // b


def _sc_degree(np_, ch):
  mesh = plsc.VectorSubcoreMesh(core_axis_name="c", subcore_axis_name="s")
  rpt = np_ // NS

  @functools.partial(
      pl.kernel,
      out_type=jax.ShapeDtypeStruct((NC, np_, DW), jnp.float32),
      mesh=mesh,
      compiler_params=pltpu.CompilerParams(use_tc_tiling_on_sc=False),
      scratch_types=[
          pltpu.VMEM_SHARED((np_, DW), jnp.float32),
          pltpu.VMEM((K2, DW), jnp.float32),
          pltpu.VMEM((2, 2, K2), jnp.int32),
      ] + [pltpu.SemaphoreType.DMA] * 4,
  )
  def deg_kernel(sd_hbm, ones_hbm, zeros_hbm, out_hbm, acc, ones_v, idxv,
                 *sems):
    isems = sems[0:2]
    ssems = sems[2:4]
    c = lax.axis_index("c")
    s = lax.axis_index("s")
    wid = c * NS + s
    pltpu.sync_copy(zeros_hbm.at[pl.ds(s * rpt, rpt)],
                    acc.at[pl.ds(s * rpt, rpt)])
    pltpu.sync_copy(ones_hbm, ones_v)
    plsc.subcore_barrier()

    @pl.loop(0, ch // 2)
    def _(t):
      idd = [
          pltpu.async_copy(sd_hbm.at[wid, 2 * t + b], idxv.at[b], isems[b])
          for b in range(2)
      ]
      sd = []
      for b in range(2):
        idd[b].wait()
        sd.append(
            pltpu.async_copy(ones_v, acc.at[idxv.at[b, 1]], ssems[b],
                             add=True))
      for b in range(2):
        sd[b].wait()

    plsc.subcore_barrier()
    pltpu.sync_copy(acc.at[pl.ds(s * rpt, rpt)],
                    out_hbm.at[c, pl.ds(s * rpt, rpt)])

  return deg_kernel


def _sc_aggregate(np_, d, cha, chb, chmax, dtype):
  del chmax
  mesh = plsc.VectorSubcoreMesh(core_axis_name="c", subcore_axis_name="s")
  rpt = np_ // NS

  @functools.partial(
      pl.kernel,
      out_type=jax.ShapeDtypeStruct((NC, np_, d), dtype),
      mesh=mesh,
      compiler_params=pltpu.CompilerParams(use_tc_tiling_on_sc=False),
      scratch_types=[
          pltpu.VMEM_SHARED((np_, d), dtype),
          pltpu.VMEM((2, K2, d), dtype),
          pltpu.VMEM((2, 2, K2), jnp.int32),
      ] + [pltpu.SemaphoreType.DMA] * 6,
  )
  def agg_kernel(table_hbm, sd_hbm, zeros_hbm, out_hbm, acc, rows, idxv,
                 *sems):
    isems = sems[0:2]
    gsems = sems[2:4]
    ssems = sems[4:6]
    c = lax.axis_index("c")
    s = lax.axis_index("s")
    wid = c * NS + s
    nch = jnp.where(c == 0, cha, chb)
    pltpu.sync_copy(zeros_hbm.at[pl.ds(s * rpt, rpt)],
                    acc.at[pl.ds(s * rpt, rpt)])
    plsc.subcore_barrier()

    @pl.loop(0, nch // 2)
    def _(t):
      idd = [
          pltpu.async_copy(sd_hbm.at[wid, 2 * t + b], idxv.at[b], isems[b])
          for b in range(2)
      ]
      gd = []
      for b in range(2):
        idd[b].wait()
        gd.append(
            pltpu.async_copy(table_hbm.at[idxv.at[b, 0]], rows.at[b],
                             gsems[b]))
      sd = []
      for b in range(2):
        gd[b].wait()
        sd.append(
            pltpu.async_copy(rows.at[b], acc.at[idxv.at[b, 1]], ssems[b],
                             add=True))
      for b in range(2):
        sd[b].wait()

    plsc.subcore_barrier()
    pltpu.sync_copy(acc.at[pl.ds(s * rpt, rpt)],
                    out_hbm.at[c, pl.ds(s * rpt, rpt)])

  return agg_kernel


def _tc_scale_body(x_ref, t0_ref, t1_ref, xs_ref, dinv_ref):
  deg = 1.0 + t0_ref[:, 0:1] + t1_ref[:, 0:1]
  dinv = lax.rsqrt(deg)
  xs_ref[...] = (x_ref[...] * dinv).astype(xs_ref.dtype)
  dinv_ref[...] = dinv


def _tc_dense_body(p0_ref, p1_ref, xs_ref, dinv_ref, w1_ref, b1_ref, w2_ref,
                   ys_ref):
  d = dinv_ref[...]
  agg = (p0_ref[...].astype(jnp.float32) + p1_ref[...].astype(jnp.float32) +
         xs_ref[...].astype(jnp.float32))
  z = d * agg
  h = jnp.dot(z, w1_ref[...], preferred_element_type=jnp.float32)
  h = jnp.maximum(h + b1_ref[...], 0.0)
  y = jnp.dot(h, w2_ref[...], preferred_element_type=jnp.float32)
  ys_ref[...] = (d * y).astype(ys_ref.dtype)


def _tc_softmax_body(c_valid, q0_ref, q1_ref, ys_ref, dinv_ref, b2_ref, o_ref):
  agg = (q0_ref[...].astype(jnp.float32) + q1_ref[...].astype(jnp.float32) +
         ys_ref[...].astype(jnp.float32))
  u = dinv_ref[...] * agg + b2_ref[...]
  col = lax.broadcasted_iota(jnp.int32, u.shape, 1)
  valid = col < c_valid
  um = jnp.where(valid, u, -jnp.inf)
  mx = jnp.max(um, axis=1, keepdims=True)
  ex = jnp.where(valid, jnp.exp(u - mx), 0.0)
  o_ref[...] = (u - mx) - jnp.log(jnp.sum(ex, axis=1, keepdims=True))


def _row_spec(d):
  return pl.BlockSpec((BN, d), lambda i: (i, 0))


def _full_spec(r, c):
  return pl.BlockSpec((r, c), lambda i: (0, 0))


def kernel(x, edge_index, W1, b1, W2, b2):
  n, f_in = x.shape
  hid = W1.shape[1]
  c_out = W2.shape[1]
  e = edge_index.shape[1]

  np_ = _cdiv(n, NS * BN) * NS * BN
  ch = _cdiv(_cdiv(e, NW * K2), 2) * 2
  ep = NW * ch * K2
  d2 = _cdiv(c_out, 64) * 64

  src = edge_index[0].astype(jnp.int32)
  dst = edge_index[1].astype(jnp.int32)
  pad = ep - e
  src_f = jnp.concatenate([src, jnp.zeros((pad,), jnp.int32)])
  dst_f = jnp.concatenate([dst, jnp.full((pad,), n, jnp.int32)])
  sd_p = jnp.stack(
      [src_f.reshape(NW, ch, K2), dst_f.reshape(NW, ch, K2)], axis=2)

  cha = max(2, (_cdiv(2 * ch, 4) // 2) * 2 - 2)
  chb = 2 * ch - cha
  chmax = max(cha, chb)
  ea = NS * cha * K2
  sd0 = jnp.stack([src_f[:ea].reshape(NS, cha, K2),
                   dst_f[:ea].reshape(NS, cha, K2)], axis=2)
  sd0 = jnp.pad(sd0, ((0, 0), (0, chmax - cha), (0, 0), (0, 0)))
  sd1 = jnp.stack([src_f[ea:].reshape(NS, chb, K2),
                   dst_f[ea:].reshape(NS, chb, K2)], axis=2)
  sd1 = jnp.pad(sd1, ((0, 0), (0, chmax - chb), (0, 0), (0, 0)))
  sd_q = jnp.concatenate([sd0, sd1], axis=0)

  x_p = jnp.zeros((np_, f_in), jnp.float32).at[:n].set(x)
  ones_rows = jnp.zeros((K2, DW), jnp.float32).at[:, 0].set(1.0)
  zeros_deg = jnp.zeros((np_, DW), jnp.float32)
  zeros_g = jnp.zeros((np_, f_in), jnp.bfloat16)
  zeros_c = jnp.zeros((np_, d2), jnp.bfloat16)
  w2_p = jnp.zeros((hid, d2), jnp.float32).at[:, :c_out].set(W2)
  b1_r = b1.reshape(1, hid)
  b2_p = jnp.zeros((1, d2), jnp.float32).at[0, :c_out].set(b2)

  deg_t = _sc_degree(np_, ch)(sd_p, ones_rows, zeros_deg)

  grid = (np_ // BN,)
  xs, dinv = pl.pallas_call(
      _tc_scale_body,
      grid=grid,
      in_specs=[_row_spec(f_in), _row_spec(DW), _row_spec(DW)],
      out_specs=[_row_spec(f_in), _row_spec(1)],
      out_shape=[
          jax.ShapeDtypeStruct((np_, f_in), jnp.bfloat16),
          jax.ShapeDtypeStruct((np_, 1), jnp.float32),
      ],
  )(x_p, deg_t[0], deg_t[1])

  p_t = _sc_aggregate(np_, f_in, cha, chb, chmax, jnp.bfloat16)(
      xs, sd_q, zeros_g)

  ys = pl.pallas_call(
      _tc_dense_body,
      grid=grid,
      in_specs=[
          _row_spec(f_in), _row_spec(f_in), _row_spec(f_in), _row_spec(1),
          _full_spec(f_in, hid), _full_spec(1, hid), _full_spec(hid, d2),
      ],
      out_specs=_row_spec(d2),
      out_shape=jax.ShapeDtypeStruct((np_, d2), jnp.bfloat16),
  )(p_t[0], p_t[1], xs, dinv, W1, b1_r, w2_p)

  q_t = _sc_aggregate(np_, d2, cha, chb, chmax, jnp.bfloat16)(
      ys, sd_q, zeros_c)

  o = pl.pallas_call(
      functools.partial(_tc_softmax_body, c_out),
      grid=grid,
      in_specs=[
          _row_spec(d2), _row_spec(d2), _row_spec(d2), _row_spec(1),
          _full_spec(1, d2),
      ],
      out_specs=_row_spec(d2),
      out_shape=jax.ShapeDtypeStruct((np_, d2), jnp.float32),
  )(q_t[0], q_t[1], ys, dinv, b2_p)

  return o[:n, :c_out]
